# baseline (speedup 1.0000x reference)
import jax
import jax.numpy as jnp
from jax import lax
from jax.experimental import pallas as pl
from jax.experimental.pallas import tpu as pltpu
from jax.experimental.pallas import tpu_sc as plsc

N_NODES = 10000
N_EDGES = 160000
D_IN = 256
D_HID = 128
N_CLASSES = 64

NC = 2
NS = 16
NT = NC * NS
N_PAD = 10240
NPT = N_PAD // NS
CW = 64
NB = 4
NH = 2
CPH = 40
E_PAD = NT * NH * CPH * CW
N_FAKE = E_PAD - N_EDGES

_mesh = plsc.VectorSubcoreMesh(core_axis_name="c", subcore_axis_name="s")


def _deg_body(src_hbm, dst_hbm, out_hbm, sidx, didx, buf, acc_out, acc_in, sem):
    c = lax.axis_index("c")
    s = lax.axis_index("s")
    for q in range(NPT // 16):
        buf[pl.ds(q * 16, 16)] = jnp.zeros((16,), jnp.float32)
    pltpu.sync_copy(buf, acc_out.at[pl.ds(s * NPT, NPT)])
    pltpu.sync_copy(buf, acc_in.at[pl.ds(s * NPT, NPT)])
    for q in range(CW // 16):
        buf[pl.ds(q * 16, 16)] = jnp.ones((16,), jnp.float32)
    plsc.subcore_barrier()

    wid = c * NS + s
    pltpu.sync_copy(src_hbm.at[wid], sidx)
    pltpu.sync_copy(dst_hbm.at[wid], didx)

    ones = buf.at[pl.ds(0, CW)]

    def fire(j, carry):
        h = j // CPH
        r = j - h * CPH
        pltpu.async_copy(ones, acc_out.at[sidx.at[h, r]], sem, add=True)
        pltpu.async_copy(ones, acc_in.at[didx.at[h, r]], sem, add=True)
        return carry

    lax.fori_loop(0, NH * CPH, fire, 0)

    def drain(j, carry):
        pltpu.make_async_copy(ones, acc_out.at[sidx.at[0, 0]], sem).wait()
        pltpu.make_async_copy(ones, acc_in.at[didx.at[0, 0]], sem).wait()
        return carry

    lax.fori_loop(0, NH * CPH, drain, 0)
    plsc.subcore_barrier()
    pltpu.sync_copy(acc_out.at[pl.ds(s * NPT, NPT)], out_hbm.at[c, 0, pl.ds(s * NPT, NPT)])
    pltpu.sync_copy(acc_in.at[pl.ds(s * NPT, NPT)], out_hbm.at[c, 1, pl.ds(s * NPT, NPT)])


_deg_call = pl.kernel(
    _deg_body,
    out_type=jax.ShapeDtypeStruct((NC, 2, N_PAD), jnp.float32),
    mesh=_mesh,
    scratch_types=[
        pltpu.VMEM((NH, CPH, CW), jnp.int32),
        pltpu.VMEM((NH, CPH, CW), jnp.int32),
        pltpu.VMEM((NPT,), jnp.float32),
        pltpu.VMEM_SHARED((N_PAD,), jnp.float32),
        pltpu.VMEM_SHARED((N_PAD,), jnp.float32),
        pltpu.SemaphoreType.DMA,
    ],
)


def _make_agg(D):
    def _agg_body(y_hbm, src_hbm, dst_hbm, out_hbm, ibs, ibd, acc, *rest):
        bufs = rest[:NB]
        gsem = rest[NB:2 * NB]
        ssem = rest[2 * NB:3 * NB]
        c = lax.axis_index("c")
        s = lax.axis_index("s")
        wid = c * NS + s

        for r in range(16):
            for q in range(D // 16):
                bufs[0][r, pl.ds(q * 16, 16)] = jnp.zeros((16,), jnp.float32)
        zsrc = bufs[0].at[pl.ds(0, 16), :]

        def zfire(k, carry):
            pltpu.async_copy(zsrc, acc.at[pl.ds(s * NPT + k * 16, 16), :], gsem[0])
            return carry

        lax.fori_loop(0, NPT // 16, zfire, 0)

        def zdrain(k, carry):
            pltpu.make_async_copy(zsrc, acc.at[pl.ds(s * NPT, 16), :], gsem[0]).wait()
            return carry

        lax.fori_loop(0, NPT // 16, zdrain, 0)
        plsc.subcore_barrier()

        def start_gather(b, j):
            pltpu.async_copy(y_hbm.at[ibs.at[j]], bufs[b], gsem[b])

        def wait_gather(b, j):
            pltpu.make_async_copy(y_hbm.at[ibs.at[j]], bufs[b], gsem[b]).wait()

        def start_scatter(b, j):
            pltpu.async_copy(bufs[b], acc.at[ibd.at[j]], ssem[b], add=True)

        def wait_scatter(b, j):
            pltpu.make_async_copy(bufs[b], acc.at[ibd.at[j]], ssem[b]).wait()

        for h in range(NH):
            pltpu.sync_copy(src_hbm.at[wid, h], ibs)
            pltpu.sync_copy(dst_hbm.at[wid, h], ibd)
            start_gather(0, 0)
            start_gather(1, 1)
            wait_gather(0, 0); start_scatter(0, 0); start_gather(2, 2)
            wait_gather(1, 1); start_scatter(1, 1); start_gather(3, 3)
            wait_gather(2, 2); start_scatter(2, 2); wait_scatter(0, 0); start_gather(0, 4)
            wait_gather(3, 3); start_scatter(3, 3); wait_scatter(1, 1); start_gather(1, 5)

            def body(g, carry):
                base = g * NB
                wait_gather(0, base + 0); start_scatter(0, base + 0); wait_scatter(2, base - 2); start_gather(2, base + 2)
                wait_gather(1, base + 1); start_scatter(1, base + 1); wait_scatter(3, base - 1); start_gather(3, base + 3)
                wait_gather(2, base + 2); start_scatter(2, base + 2); wait_scatter(0, base + 0); start_gather(0, base + 4)
                wait_gather(3, base + 3); start_scatter(3, base + 3); wait_scatter(1, base + 1); start_gather(1, base + 5)
                return carry

            lax.fori_loop(1, CPH // NB - 1, body, 0)

            base = CPH - NB
            wait_gather(0, base + 0); start_scatter(0, base + 0); wait_scatter(2, base - 2); start_gather(2, base + 2)
            wait_gather(1, base + 1); start_scatter(1, base + 1); wait_scatter(3, base - 1); start_gather(3, base + 3)
            wait_gather(2, base + 2); start_scatter(2, base + 2); wait_scatter(0, base + 0)
            wait_gather(3, base + 3); start_scatter(3, base + 3); wait_scatter(1, base + 1)
            wait_scatter(2, base + 2)
            wait_scatter(3, base + 3)

        plsc.subcore_barrier()
        pltpu.sync_copy(acc.at[pl.ds(s * NPT, NPT), :],
                        out_hbm.at[c, pl.ds(s * NPT, NPT), :])

    return pl.kernel(
        _agg_body,
        out_type=jax.ShapeDtypeStruct((NC, N_PAD, D), jnp.float32),
        mesh=_mesh,
        scratch_types=[
            pltpu.VMEM((CPH, CW), jnp.int32),
            pltpu.VMEM((CPH, CW), jnp.int32),
            pltpu.VMEM_SHARED((N_PAD, D), jnp.float32),
        ] + [pltpu.VMEM((CW, D), jnp.float32) for _ in range(NB)]
          + [pltpu.SemaphoreType.DMA for _ in range(2 * NB)],
    )


_agg_hid = _make_agg(D_HID)


def _norm(d):
    return jnp.where(d > 0.0, lax.rsqrt(jnp.maximum(d, 1.0)), 0.0)


def _d_out(dp_ref):
    d = dp_ref[0, 0, :N_NODES] + dp_ref[1, 0, :N_NODES]
    row = lax.broadcasted_iota(jnp.int32, (N_NODES,), 0)
    return d - jnp.where(row == 0, jnp.float32(N_FAKE), jnp.float32(0.0))


def _y1_body(x_ref, w_ref, dp_ref, o_ref):
    ns = _norm(_d_out(dp_ref))
    z = jnp.dot(x_ref[...], w_ref[...], preferred_element_type=jnp.float32)
    o_ref[...] = z * ns[:, None]


def _y2_body(a_ref, dp_ref, b1_ref, w_ref, o_ref):
    a = a_ref[0, :N_NODES, :] + a_ref[1, :N_NODES, :]
    nd = _norm(dp_ref[0, 1, :N_NODES] + dp_ref[1, 1, :N_NODES])
    ns = _norm(_d_out(dp_ref))
    h = jnp.maximum(a * nd[:, None] + b1_ref[...][None, :], 0.0)
    o_ref[...] = jnp.dot(h * ns[:, None], w_ref[...],
                         preferred_element_type=jnp.float32)


def _out_body(a_ref, dp_ref, b2_ref, o_ref):
    a = a_ref[0, :N_NODES, :N_CLASSES] + a_ref[1, :N_NODES, :N_CLASSES]
    nd = _norm(dp_ref[0, 1, :N_NODES] + dp_ref[1, 1, :N_NODES])
    o_ref[...] = a * nd[:, None] + b2_ref[...][None, :]


def _tc_call(body, out_shape):
    return pl.pallas_call(body, out_shape=jax.ShapeDtypeStruct(out_shape, jnp.float32))


@jax.jit
def kernel(features, edge_index, W1, b1, W2, b2):
    src = jnp.concatenate(
        [edge_index[0].astype(jnp.int32), jnp.zeros((N_FAKE,), jnp.int32)]
    ).reshape(NT, NH, CPH, CW)
    dst = jnp.concatenate(
        [edge_index[1].astype(jnp.int32), jnp.full((N_FAKE,), N_NODES, jnp.int32)]
    ).reshape(NT, NH, CPH, CW)

    W2p = jnp.zeros((D_HID, D_HID), jnp.float32).at[:, :N_CLASSES].set(W2)

    dp = _deg_call(src, dst)
    y1 = _tc_call(_y1_body, (N_NODES, D_HID))(features, W1, dp)
    p1 = _agg_hid(y1, src, dst)
    y2 = _tc_call(_y2_body, (N_NODES, D_HID))(p1, dp, b1, W2p)
    p2 = _agg_hid(y2, src, dst)
    out = _tc_call(_out_body, (N_NODES, N_CLASSES))(p2, dp, b2)
    return out


# speedup vs baseline: 1.0001x; 1.0001x over previous
import jax
import jax.numpy as jnp
from jax import lax
from jax.experimental import pallas as pl
from jax.experimental.pallas import tpu as pltpu
from jax.experimental.pallas import tpu_sc as plsc

N_NODES = 10000
N_EDGES = 160000
D_IN = 256
D_HID = 128
N_CLASSES = 64

NC = 2
NS = 16
NT = NC * NS
N_PAD = 10240
NPT = N_PAD // NS
CW = 64
NB = 4
NH = 2
CPH = 40
E_PAD = NT * NH * CPH * CW
N_FAKE = E_PAD - N_EDGES

_mesh = plsc.VectorSubcoreMesh(core_axis_name="c", subcore_axis_name="s")


def _deg_body(src_hbm, dst_hbm, out_hbm, sidx, didx, buf, acc_out, acc_in, sem):
    c = lax.axis_index("c")
    s = lax.axis_index("s")
    for q in range(NPT // 16):
        buf[pl.ds(q * 16, 16)] = jnp.zeros((16,), jnp.float32)
    pltpu.sync_copy(buf, acc_out.at[pl.ds(s * NPT, NPT)])
    pltpu.sync_copy(buf, acc_in.at[pl.ds(s * NPT, NPT)])
    for q in range(CW // 16):
        buf[pl.ds(q * 16, 16)] = jnp.ones((16,), jnp.float32)
    plsc.subcore_barrier()

    wid = c * NS + s
    pltpu.sync_copy(src_hbm.at[wid], sidx)
    pltpu.sync_copy(dst_hbm.at[wid], didx)

    ones = buf.at[pl.ds(0, CW)]

    def fire(j, carry):
        h = j // CPH
        r = j - h * CPH
        pltpu.async_copy(ones, acc_out.at[sidx.at[h, r]], sem, add=True)
        pltpu.async_copy(ones, acc_in.at[didx.at[h, r]], sem, add=True)
        return carry

    lax.fori_loop(0, NH * CPH, fire, 0)

    def drain(j, carry):
        pltpu.make_async_copy(ones, acc_out.at[sidx.at[0, 0]], sem).wait()
        pltpu.make_async_copy(ones, acc_in.at[didx.at[0, 0]], sem).wait()
        return carry

    lax.fori_loop(0, NH * CPH, drain, 0)
    plsc.subcore_barrier()
    pltpu.sync_copy(acc_out.at[pl.ds(s * NPT, NPT)], out_hbm.at[c, 0, pl.ds(s * NPT, NPT)])
    pltpu.sync_copy(acc_in.at[pl.ds(s * NPT, NPT)], out_hbm.at[c, 1, pl.ds(s * NPT, NPT)])


_deg_call = pl.kernel(
    _deg_body,
    out_type=jax.ShapeDtypeStruct((NC, 2, N_PAD), jnp.float32),
    mesh=_mesh,
    scratch_types=[
        pltpu.VMEM((NH, CPH, CW), jnp.int32),
        pltpu.VMEM((NH, CPH, CW), jnp.int32),
        pltpu.VMEM((NPT,), jnp.float32),
        pltpu.VMEM_SHARED((N_PAD,), jnp.float32),
        pltpu.VMEM_SHARED((N_PAD,), jnp.float32),
        pltpu.SemaphoreType.DMA,
    ],
)


def _make_agg(D):
    def _agg_body(y_hbm, src_hbm, dst_hbm, out_hbm, ibs, ibd, acc, *rest):
        bufs = rest[:NB]
        gsem = rest[NB:2 * NB]
        ssem = rest[2 * NB:3 * NB]
        c = lax.axis_index("c")
        s = lax.axis_index("s")
        wid = c * NS + s

        for r in range(16):
            for q in range(D // 16):
                bufs[0][r, pl.ds(q * 16, 16)] = jnp.zeros((16,), jnp.float32)
        zsrc = bufs[0].at[pl.ds(0, 16), :]

        def zfire(k, carry):
            pltpu.async_copy(zsrc, acc.at[pl.ds(s * NPT + k * 16, 16), :], gsem[0])
            return carry

        lax.fori_loop(0, NPT // 16, zfire, 0)

        def zdrain(k, carry):
            pltpu.make_async_copy(zsrc, acc.at[pl.ds(s * NPT, 16), :], gsem[0]).wait()
            return carry

        lax.fori_loop(0, NPT // 16, zdrain, 0)
        plsc.subcore_barrier()

        def start_gather(b, j):
            pltpu.async_copy(y_hbm.at[ibs.at[j]], bufs[b], gsem[b])

        def wait_gather(b, j):
            pltpu.make_async_copy(y_hbm.at[ibs.at[j]], bufs[b], gsem[b]).wait()

        def start_scatter(b, j):
            pltpu.async_copy(bufs[b], acc.at[ibd.at[j]], ssem[b], add=True)

        def wait_scatter(b, j):
            pltpu.make_async_copy(bufs[b], acc.at[ibd.at[j]], ssem[b]).wait()

        for h in range(NH):
            pltpu.sync_copy(src_hbm.at[wid, h], ibs)
            pltpu.sync_copy(dst_hbm.at[wid, h], ibd)
            start_gather(0, 0)
            start_gather(1, 1)
            wait_gather(0, 0); start_scatter(0, 0); start_gather(2, 2)
            wait_gather(1, 1); start_scatter(1, 1); start_gather(3, 3)
            wait_gather(2, 2); start_scatter(2, 2); wait_scatter(0, 0); start_gather(0, 4)
            wait_gather(3, 3); start_scatter(3, 3); wait_scatter(1, 1); start_gather(1, 5)

            def body(g, carry):
                base = g * NB
                wait_gather(0, base + 0); start_scatter(0, base + 0); wait_scatter(2, base - 2); start_gather(2, base + 2)
                wait_gather(1, base + 1); start_scatter(1, base + 1); wait_scatter(3, base - 1); start_gather(3, base + 3)
                wait_gather(2, base + 2); start_scatter(2, base + 2); wait_scatter(0, base + 0); start_gather(0, base + 4)
                wait_gather(3, base + 3); start_scatter(3, base + 3); wait_scatter(1, base + 1); start_gather(1, base + 5)
                return carry

            lax.fori_loop(1, CPH // NB - 1, body, 0)

            base = CPH - NB
            wait_gather(0, base + 0); start_scatter(0, base + 0); wait_scatter(2, base - 2); start_gather(2, base + 2)
            wait_gather(1, base + 1); start_scatter(1, base + 1); wait_scatter(3, base - 1); start_gather(3, base + 3)
            wait_gather(2, base + 2); start_scatter(2, base + 2); wait_scatter(0, base + 0)
            wait_gather(3, base + 3); start_scatter(3, base + 3); wait_scatter(1, base + 1)
            wait_scatter(2, base + 2)
            wait_scatter(3, base + 3)

        plsc.subcore_barrier()
        pltpu.sync_copy(acc.at[pl.ds(s * NPT, NPT), :],
                        out_hbm.at[c, pl.ds(s * NPT, NPT), :])

    return pl.kernel(
        _agg_body,
        out_type=jax.ShapeDtypeStruct((NC, N_PAD, D), jnp.float32),
        mesh=_mesh,
        scratch_types=[
            pltpu.VMEM((CPH, CW), jnp.int32),
            pltpu.VMEM((CPH, CW), jnp.int32),
            pltpu.VMEM_SHARED((N_PAD, D), jnp.float32),
        ] + [pltpu.VMEM((CW, D), jnp.float32) for _ in range(NB)]
          + [pltpu.SemaphoreType.DMA for _ in range(2 * NB)],
    )


_agg_hid = _make_agg(D_HID)


def _norm(d):
    return jnp.where(d > 0.0, lax.rsqrt(jnp.maximum(d, 1.0)), 0.0)


def _d_out(dp_ref):
    d = dp_ref[0, 0, :N_NODES] + dp_ref[1, 0, :N_NODES]
    row = lax.broadcasted_iota(jnp.int32, (N_NODES,), 0)
    return d - jnp.where(row == 0, jnp.float32(N_FAKE), jnp.float32(0.0))


def _y1_body(x_ref, w_ref, dp_ref, o_ref):
    ns = _norm(_d_out(dp_ref))
    z = jnp.dot(x_ref[...], w_ref[...], preferred_element_type=jnp.float32)
    o_ref[...] = z * ns[:, None]


def _y2_body(a_ref, dp_ref, b1_ref, w_ref, o_ref):
    a = a_ref[0, :N_NODES, :] + a_ref[1, :N_NODES, :]
    nd = _norm(dp_ref[0, 1, :N_NODES] + dp_ref[1, 1, :N_NODES])
    ns = _norm(_d_out(dp_ref))
    h = jnp.maximum(a * nd[:, None] + b1_ref[...][None, :], 0.0)
    o_ref[...] = jnp.dot(h * ns[:, None], w_ref[...],
                         preferred_element_type=jnp.float32)


def _out_body(a_ref, dp_ref, b2_ref, o_ref):
    a = a_ref[0, :N_NODES, :N_CLASSES] + a_ref[1, :N_NODES, :N_CLASSES]
    nd = _norm(dp_ref[0, 1, :N_NODES] + dp_ref[1, 1, :N_NODES])
    o_ref[...] = a * nd[:, None] + b2_ref[...][None, :]


def _tc_call(body, out_shape):
    return pl.pallas_call(body, out_shape=jax.ShapeDtypeStruct(out_shape, jnp.float32))


@jax.jit
def kernel(features, edge_index, W1, b1, W2, b2):
    src = jnp.concatenate(
        [edge_index[0].astype(jnp.int32), jnp.zeros((N_FAKE,), jnp.int32)]
    ).reshape(NT, NH, CPH, CW)
    fake_dst = N_NODES + jnp.arange(N_FAKE, dtype=jnp.int32) % (N_PAD - N_NODES)
    dst = jnp.concatenate(
        [edge_index[1].astype(jnp.int32), fake_dst]
    ).reshape(NT, NH, CPH, CW)

    W2p = jnp.zeros((D_HID, D_HID), jnp.float32).at[:, :N_CLASSES].set(W2)

    dp = _deg_call(src, dst)
    y1 = _tc_call(_y1_body, (N_NODES, D_HID))(features, W1, dp)
    p1 = _agg_hid(y1, src, dst)
    y2 = _tc_call(_y2_body, (N_NODES, D_HID))(p1, dp, b1, W2p)
    p2 = _agg_hid(y2, src, dst)
    out = _tc_call(_out_body, (N_NODES, N_CLASSES))(p2, dp, b2)
    return out


# speedup vs baseline: 2.8905x; 2.8902x over previous
import jax
import jax.numpy as jnp
from jax import lax
from jax.experimental import pallas as pl
from jax.experimental.pallas import tpu as pltpu
from jax.experimental.pallas import tpu_sc as plsc

N_NODES = 10000
N_EDGES = 160000
D_IN = 256
D_HID = 128
N_CLASSES = 64

NC = 2
NS = 16
NT = NC * NS
N_PAD = 10240
NPT = N_PAD // NS
CW = 64
NB = 4
NH = 2
CPH = 40
E_PAD = NT * NH * CPH * CW
N_FAKE = E_PAD - N_EDGES

_mesh = plsc.VectorSubcoreMesh(core_axis_name="c", subcore_axis_name="s")


def _deg_body(src_hbm, dst_hbm, out_hbm, sidx, didx, buf, acc_out, acc_in, sem):
    c = lax.axis_index("c")
    s = lax.axis_index("s")
    for q in range(NPT // 16):
        buf[pl.ds(q * 16, 16)] = jnp.zeros((16,), jnp.float32)
    pltpu.sync_copy(buf, acc_out.at[pl.ds(s * NPT, NPT)])
    pltpu.sync_copy(buf, acc_in.at[pl.ds(s * NPT, NPT)])
    for q in range(CW // 16):
        buf[pl.ds(q * 16, 16)] = jnp.ones((16,), jnp.float32)
    plsc.subcore_barrier()

    wid = c * NS + s
    pltpu.sync_copy(src_hbm.at[wid], sidx)
    pltpu.sync_copy(dst_hbm.at[wid], didx)

    ones = buf.at[pl.ds(0, CW)]

    def fire(j, carry):
        h = j // CPH
        r = j - h * CPH
        pltpu.async_copy(ones, acc_out.at[sidx.at[h, r]], sem, add=True)
        pltpu.async_copy(ones, acc_in.at[didx.at[h, r]], sem, add=True)
        return carry

    lax.fori_loop(0, NH * CPH, fire, 0)

    def drain(j, carry):
        pltpu.make_async_copy(ones, acc_out.at[sidx.at[0, 0]], sem).wait()
        pltpu.make_async_copy(ones, acc_in.at[didx.at[0, 0]], sem).wait()
        return carry

    lax.fori_loop(0, NH * CPH, drain, 0)
    plsc.subcore_barrier()
    pltpu.sync_copy(acc_out.at[pl.ds(s * NPT, NPT)], out_hbm.at[c, 0, pl.ds(s * NPT, NPT)])
    pltpu.sync_copy(acc_in.at[pl.ds(s * NPT, NPT)], out_hbm.at[c, 1, pl.ds(s * NPT, NPT)])


_deg_call = pl.kernel(
    _deg_body,
    out_type=jax.ShapeDtypeStruct((NC, 2, N_PAD), jnp.float32),
    mesh=_mesh,
    scratch_types=[
        pltpu.VMEM((NH, CPH, CW), jnp.int32),
        pltpu.VMEM((NH, CPH, CW), jnp.int32),
        pltpu.VMEM((NPT,), jnp.float32),
        pltpu.VMEM_SHARED((N_PAD,), jnp.float32),
        pltpu.VMEM_SHARED((N_PAD,), jnp.float32),
        pltpu.SemaphoreType.DMA,
    ],
)


def _make_agg(D):
    def _agg_body(y_hbm, src_hbm, dst_hbm, out_hbm, ibs, ibd, acc, *rest):
        bufs = rest[:NB]
        gsem = rest[NB:2 * NB]
        ssem = rest[2 * NB:3 * NB]
        c = lax.axis_index("c")
        s = lax.axis_index("s")
        wid = c * NS + s

        for r in range(16):
            for q in range(D // 16):
                bufs[0][r, pl.ds(q * 16, 16)] = jnp.zeros((16,), jnp.float32)
        zsrc = bufs[0].at[pl.ds(0, 16), :]

        def zfire(k, carry):
            pltpu.async_copy(zsrc, acc.at[pl.ds(s * NPT + k * 16, 16), :], gsem[0])
            return carry

        lax.fori_loop(0, NPT // 16, zfire, 0)

        def zdrain(k, carry):
            pltpu.make_async_copy(zsrc, acc.at[pl.ds(s * NPT, 16), :], gsem[0]).wait()
            return carry

        lax.fori_loop(0, NPT // 16, zdrain, 0)
        plsc.subcore_barrier()

        def start_gather(b, j):
            pltpu.async_copy(y_hbm.at[ibs.at[j]], bufs[b], gsem[b])

        def wait_gather(b, j):
            pltpu.make_async_copy(y_hbm.at[ibs.at[j]], bufs[b], gsem[b]).wait()

        def start_scatter(b, j):
            pltpu.async_copy(bufs[b], acc.at[ibd.at[j]], ssem[b], add=True)

        def wait_scatter(b, j):
            pltpu.make_async_copy(bufs[b], acc.at[ibd.at[j]], ssem[b]).wait()

        for h in range(NH):
            pltpu.sync_copy(src_hbm.at[wid, h], ibs)
            pltpu.sync_copy(dst_hbm.at[wid, h], ibd)
            start_gather(0, 0)
            start_gather(1, 1)
            wait_gather(0, 0); start_scatter(0, 0); start_gather(2, 2)
            wait_gather(1, 1); start_scatter(1, 1); start_gather(3, 3)
            wait_gather(2, 2); start_scatter(2, 2); wait_scatter(0, 0); start_gather(0, 4)
            wait_gather(3, 3); start_scatter(3, 3); wait_scatter(1, 1); start_gather(1, 5)

            def body(g, carry):
                base = g * NB
                wait_gather(0, base + 0); start_scatter(0, base + 0); wait_scatter(2, base - 2); start_gather(2, base + 2)
                wait_gather(1, base + 1); start_scatter(1, base + 1); wait_scatter(3, base - 1); start_gather(3, base + 3)
                wait_gather(2, base + 2); start_scatter(2, base + 2); wait_scatter(0, base + 0); start_gather(0, base + 4)
                wait_gather(3, base + 3); start_scatter(3, base + 3); wait_scatter(1, base + 1); start_gather(1, base + 5)
                return carry

            lax.fori_loop(1, CPH // NB - 1, body, 0)

            base = CPH - NB
            wait_gather(0, base + 0); start_scatter(0, base + 0); wait_scatter(2, base - 2); start_gather(2, base + 2)
            wait_gather(1, base + 1); start_scatter(1, base + 1); wait_scatter(3, base - 1); start_gather(3, base + 3)
            wait_gather(2, base + 2); start_scatter(2, base + 2); wait_scatter(0, base + 0)
            wait_gather(3, base + 3); start_scatter(3, base + 3); wait_scatter(1, base + 1)
            wait_scatter(2, base + 2)
            wait_scatter(3, base + 3)

        plsc.subcore_barrier()
        pltpu.sync_copy(acc.at[pl.ds(s * NPT, NPT), :],
                        out_hbm.at[c, pl.ds(s * NPT, NPT), :])

    return pl.kernel(
        _agg_body,
        out_type=jax.ShapeDtypeStruct((NC, N_PAD, D), jnp.float32),
        mesh=_mesh,
        scratch_types=[
            pltpu.VMEM((CPH, CW), jnp.int32),
            pltpu.VMEM((CPH, CW), jnp.int32),
            pltpu.VMEM_SHARED((N_PAD, D), jnp.float32),
        ] + [pltpu.VMEM((CW, D), jnp.float32) for _ in range(NB)]
          + [pltpu.SemaphoreType.DMA for _ in range(2 * NB)],
    )


_agg_hid = _make_agg(D_HID)


def _norm(d):
    return jnp.where(d > 0.0, lax.rsqrt(jnp.maximum(d, 1.0)), 0.0)


def _d_out(dp_ref):
    d = dp_ref[0, 0, :N_NODES] + dp_ref[1, 0, :N_NODES]
    row = lax.broadcasted_iota(jnp.int32, (N_NODES,), 0)
    return d - jnp.where(row < N_FAKE, jnp.float32(1.0), jnp.float32(0.0))


def _y1_body(x_ref, w_ref, dp_ref, o_ref):
    ns = _norm(_d_out(dp_ref))
    z = jnp.dot(x_ref[...], w_ref[...], preferred_element_type=jnp.float32)
    o_ref[...] = z * ns[:, None]


def _y2_body(a_ref, dp_ref, b1_ref, w_ref, o_ref):
    a = a_ref[0, :N_NODES, :] + a_ref[1, :N_NODES, :]
    nd = _norm(dp_ref[0, 1, :N_NODES] + dp_ref[1, 1, :N_NODES])
    ns = _norm(_d_out(dp_ref))
    h = jnp.maximum(a * nd[:, None] + b1_ref[...][None, :], 0.0)
    o_ref[...] = jnp.dot(h * ns[:, None], w_ref[...],
                         preferred_element_type=jnp.float32)


def _out_body(a_ref, dp_ref, b2_ref, o_ref):
    a = a_ref[0, :N_NODES, :N_CLASSES] + a_ref[1, :N_NODES, :N_CLASSES]
    nd = _norm(dp_ref[0, 1, :N_NODES] + dp_ref[1, 1, :N_NODES])
    o_ref[...] = a * nd[:, None] + b2_ref[...][None, :]


def _tc_call(body, out_shape):
    return pl.pallas_call(body, out_shape=jax.ShapeDtypeStruct(out_shape, jnp.float32))


@jax.jit
def kernel(features, edge_index, W1, b1, W2, b2):
    fake_src = jnp.arange(N_FAKE, dtype=jnp.int32)
    src = jnp.concatenate(
        [edge_index[0].astype(jnp.int32), fake_src]
    ).reshape(NT, NH, CPH, CW)
    fake_dst = N_NODES + jnp.arange(N_FAKE, dtype=jnp.int32) % (N_PAD - N_NODES)
    dst = jnp.concatenate(
        [edge_index[1].astype(jnp.int32), fake_dst]
    ).reshape(NT, NH, CPH, CW)

    W2p = jnp.zeros((D_HID, D_HID), jnp.float32).at[:, :N_CLASSES].set(W2)

    dp = _deg_call(src, dst)
    y1 = _tc_call(_y1_body, (N_NODES, D_HID))(features, W1, dp)
    p1 = _agg_hid(y1, src, dst)
    y2 = _tc_call(_y2_body, (N_NODES, D_HID))(p1, dp, b1, W2p)
    p2 = _agg_hid(y2, src, dst)
    out = _tc_call(_out_body, (N_NODES, N_CLASSES))(p2, dp, b2)
    return out
